# baseline (speedup 1.0000x reference)
import jax
import jax.numpy as jnp
from jax import lax
from jax.experimental import pallas as pl
from jax.experimental.pallas import tpu as pltpu
from jax.experimental.pallas import tpu_sc as plsc

_MAGIC = 8388608.0

_D = 768
_ROWS = 32 * 1024
_NC, _NS, _L = 2, 16, 16
_NW = _NC * _NS
_ROWS_W = _ROWS // _NW
_R = 16
_NB = 4
_N_CHUNKS = _ROWS_W // _R
_N_GROUPS = _N_CHUNKS // _NB


def _round16(v):
    vi = v.view(jnp.uint32)
    s = vi & jnp.uint32(0x80000000)
    a = (vi & jnp.uint32(0x7FFFFFFF)).view(jnp.float32)
    r = (a + _MAGIC) - _MAGIC
    r = jnp.where(a < _MAGIC, r, a)
    return (r.view(jnp.uint32) | s).view(jnp.float32)


def _sc_body(x_hbm, o_hbm,
             i0, i1, i2, i3, o0, o1, o2, o3,
             si0, si1, si2, si3, so0, so1, so2, so3):
    inbs, outbs = (i0, i1, i2, i3), (o0, o1, o2, o3)
    sis, sos = (si0, si1, si2, si3), (so0, so1, so2, so3)
    wid = lax.axis_index("s") * _NC + lax.axis_index("c")
    base = wid * _ROWS_W

    for b in range(_NB):
        pltpu.make_async_copy(
            x_hbm.at[pl.ds(base + b * _R, _R), :], inbs[b], sis[b]
        ).start()

    def group(g, _):
        for b in range(_NB):
            row0 = base + (g * _NB + b) * _R
            pltpu.make_async_copy(
                x_hbm.at[pl.ds(row0, _R), :], inbs[b], sis[b]
            ).wait()

            @pl.when(g > 0)
            def _():
                pltpu.make_async_copy(
                    outbs[b], o_hbm.at[pl.ds(row0 - _NB * _R, _R), :], sos[b]
                ).wait()

            inr, outr = inbs[b], outbs[b]

            @plsc.parallel_loop(0, _R * _D, step=_L, unroll=4)
            def _(i):
                r = i // _D
                c = i - r * _D
                outr[r, pl.ds(c, _L)] = _round16(inr[r, pl.ds(c, _L)])

            pltpu.make_async_copy(
                outr, o_hbm.at[pl.ds(row0, _R), :], sos[b]
            ).start()

            @pl.when(g + 1 < _N_GROUPS)
            def _():
                pltpu.make_async_copy(
                    x_hbm.at[pl.ds(row0 + _NB * _R, _R), :], inbs[b], sis[b]
                ).start()

        return 0

    lax.fori_loop(0, _N_GROUPS, group, 0)

    for b in range(_NB):
        row_last = base + (_N_CHUNKS - _NB + b) * _R
        pltpu.make_async_copy(
            outbs[b], o_hbm.at[pl.ds(row_last, _R), :], sos[b]
        ).wait()


@jax.jit
def _sc_round(x2d):
    mesh = plsc.VectorSubcoreMesh(core_axis_name="c", subcore_axis_name="s")
    vm = pltpu.VMEM((_R, _D), jnp.float32)
    f = pl.kernel(
        _sc_body,
        out_type=jax.ShapeDtypeStruct((_ROWS, _D), jnp.float32),
        mesh=mesh,
        compiler_params=pltpu.CompilerParams(
            use_tc_tiling_on_sc=True, skip_device_barrier=True
        ),
        scratch_types=[vm] * 8 + [pltpu.SemaphoreType.DMA] * 8,
    )
    return f(x2d)


def kernel(x):
    B, S, D = x.shape
    return _sc_round(x.reshape(B * S, D)).reshape(B, S, D)


# speedup vs baseline: 1.4242x; 1.4242x over previous
import jax
import jax.numpy as jnp
from jax import lax
from jax.experimental import pallas as pl
from jax.experimental.pallas import tpu as pltpu
from jax.experimental.pallas import tpu_sc as plsc

_MAGIC = 8388608.0

_D = 768
_ROWS = 32 * 1024
_NC, _NS, _L = 2, 16, 16
_NW = _NC * _NS
_ROWS_W = _ROWS // _NW
_R = 32
_NB = 4
_N_CHUNKS = _ROWS_W // _R
_N_GROUPS = _N_CHUNKS // _NB


def _round16(v):
    vi = v.view(jnp.uint32)
    s = vi & jnp.uint32(0x80000000)
    a = (vi & jnp.uint32(0x7FFFFFFF)).view(jnp.float32)
    r = (a + _MAGIC) - _MAGIC
    r = jnp.where(a < _MAGIC, r, a)
    return (r.view(jnp.uint32) | s).view(jnp.float32)


def _sc_body(x_hbm, o_hbm, b0, b1, b2, b3, si0, si1, si2, si3,
             so0, so1, so2, so3):
    bufs = (b0, b1, b2, b3)
    sis = (si0, si1, si2, si3)
    sos = (so0, so1, so2, so3)
    wid = lax.axis_index("s") * _NC + lax.axis_index("c")
    base = wid * _ROWS_W

    for b in range(2):
        pltpu.make_async_copy(
            x_hbm.at[pl.ds(base + b * _R, _R), :], bufs[b], sis[b]
        ).start()

    def group(g4, _):
        for b in range(_NB):
            g = g4 * _NB + b
            row0 = base + g * _R
            pltpu.make_async_copy(
                x_hbm.at[pl.ds(row0, _R), :], bufs[b], sis[b]
            ).wait()

            buf = bufs[b]

            @plsc.parallel_loop(0, _R * _D, step=_L, unroll=8)
            def _(i):
                r = i // _D
                c = i - r * _D
                buf[r, pl.ds(c, _L)] = _round16(buf[r, pl.ds(c, _L)])

            pltpu.make_async_copy(
                buf, o_hbm.at[pl.ds(row0, _R), :], sos[b]
            ).start()

            b2 = (b + 2) % _NB

            @pl.when(g + 2 < _N_CHUNKS)
            def _():
                @pl.when(g >= 2)
                def _():
                    pltpu.make_async_copy(
                        bufs[b2],
                        o_hbm.at[pl.ds(row0 - 2 * _R, _R), :],
                        sos[b2],
                    ).wait()

                pltpu.make_async_copy(
                    x_hbm.at[pl.ds(row0 + 2 * _R, _R), :], bufs[b2], sis[b2]
                ).start()

        return 0

    lax.fori_loop(0, _N_GROUPS, group, 0)

    for b in range(_NB):
        row_last = base + (_N_CHUNKS - _NB + b) * _R
        pltpu.make_async_copy(
            bufs[b], o_hbm.at[pl.ds(row_last, _R), :], sos[b]
        ).wait()


@jax.jit
def _sc_round(x2d):
    mesh = plsc.VectorSubcoreMesh(core_axis_name="c", subcore_axis_name="s")
    vm = pltpu.VMEM((_R, _D), jnp.float32)
    f = pl.kernel(
        _sc_body,
        out_type=jax.ShapeDtypeStruct((_ROWS, _D), jnp.float32),
        mesh=mesh,
        compiler_params=pltpu.CompilerParams(
            use_tc_tiling_on_sc=True, skip_device_barrier=True
        ),
        scratch_types=[vm] * 4 + [pltpu.SemaphoreType.DMA] * 8,
    )
    return f(x2d)


def kernel(x):
    B, S, D = x.shape
    return _sc_round(x.reshape(B * S, D)).reshape(B, S, D)


# speedup vs baseline: 1.5057x; 1.0573x over previous
import jax
import jax.numpy as jnp
from jax import lax
from jax.experimental import pallas as pl
from jax.experimental.pallas import tpu as pltpu
from jax.experimental.pallas import tpu_sc as plsc

_MAGIC = 8388608.0

_D = 768
_ROWS = 32 * 1024
_NC, _NS, _L = 2, 16, 16
_NW = _NC * _NS
_ROWS_W = _ROWS // _NW
_R = 16
_NB = 4
_N_CHUNKS = _ROWS_W // _R
_N_GROUPS = _N_CHUNKS // _NB


def _round16(v):
    vi = v.view(jnp.uint32)
    s = vi & jnp.uint32(0x80000000)
    a = (vi & jnp.uint32(0x7FFFFFFF)).view(jnp.float32)
    r = (a + _MAGIC) - _MAGIC
    r = jnp.where(a < _MAGIC, r, a)
    return (r.view(jnp.uint32) | s).view(jnp.float32)


def _sc_body(x_hbm, o_hbm,
             i0, i1, i2, i3, o0, o1, o2, o3,
             si0, si1, si2, si3, so0, so1, so2, so3):
    inbs, outbs = (i0, i1, i2, i3), (o0, o1, o2, o3)
    sis, sos = (si0, si1, si2, si3), (so0, so1, so2, so3)
    wid = lax.axis_index("s") * _NC + lax.axis_index("c")
    base = wid * _ROWS_W

    for b in range(_NB):
        pltpu.make_async_copy(
            x_hbm.at[pl.ds(base + b * _R, _R), :], inbs[b], sis[b]
        ).start()

    def group(g, _):
        for b in range(_NB):
            row0 = base + (g * _NB + b) * _R
            pltpu.make_async_copy(
                x_hbm.at[pl.ds(row0, _R), :], inbs[b], sis[b]
            ).wait()

            @pl.when(g > 0)
            def _():
                pltpu.make_async_copy(
                    outbs[b], o_hbm.at[pl.ds(row0 - _NB * _R, _R), :], sos[b]
                ).wait()

            inr, outr = inbs[b], outbs[b]

            @plsc.parallel_loop(0, _R * (_D // _L), step=1, unroll=8)
            def _(k):
                r = k & (_R - 1)
                c = (k >> 4) * _L
                outr[r, pl.ds(c, _L)] = _round16(inr[r, pl.ds(c, _L)])

            pltpu.make_async_copy(
                outr, o_hbm.at[pl.ds(row0, _R), :], sos[b]
            ).start()

            @pl.when(g + 1 < _N_GROUPS)
            def _():
                pltpu.make_async_copy(
                    x_hbm.at[pl.ds(row0 + _NB * _R, _R), :], inbs[b], sis[b]
                ).start()

        return 0

    lax.fori_loop(0, _N_GROUPS, group, 0)

    for b in range(_NB):
        row_last = base + (_N_CHUNKS - _NB + b) * _R
        pltpu.make_async_copy(
            outbs[b], o_hbm.at[pl.ds(row_last, _R), :], sos[b]
        ).wait()


@jax.jit
def _sc_round(x2d):
    mesh = plsc.VectorSubcoreMesh(core_axis_name="c", subcore_axis_name="s")
    vm = pltpu.VMEM((_R, _D), jnp.float32)
    f = pl.kernel(
        _sc_body,
        out_type=jax.ShapeDtypeStruct((_ROWS, _D), jnp.float32),
        mesh=mesh,
        compiler_params=pltpu.CompilerParams(
            use_tc_tiling_on_sc=True, skip_device_barrier=True
        ),
        scratch_types=[vm] * 8 + [pltpu.SemaphoreType.DMA] * 8,
    )
    return f(x2d)


def kernel(x):
    B, S, D = x.shape
    return _sc_round(x.reshape(B * S, D)).reshape(B, S, D)
